# baseline (speedup 1.0000x reference)
import functools

import jax
import jax.numpy as jnp
from jax import lax
from jax.experimental import pallas as pl
from jax.experimental.pallas import tpu as pltpu
from jax.experimental.pallas import tpu_sc as plsc

_N = 50000
_E = 800000
_IN_C = 32
_OUT_C = 64
_PERIODS = 8

_NC, _NS, _L = 2, 16, 16
_NW = _NC * _NS

_N_PAD = 51200
_NROWS_TILE = _N_PAD // _NS
_E2 = _E + _N
_CHUNK = 128
_E2_ROWS = 6656
_E2_PAD = _E2_ROWS * _CHUNK
_EROWS_W = _E2_ROWS // _NW
_EROWS_TILE = _E2_ROWS // _NS
_HALF = _EROWS_TILE // 2
_ZROWS = 200
_QR = 104

_N_PAD_AX = 50048
_AROWS_TILE = _N_PAD_AX // _NS
_SB = 16
_NSB = _EROWS_TILE // _SB
_AZROWS = 112
_NRB = 4

_f32 = jnp.float32
_i32 = jnp.int32


def _sc_mesh():
  return plsc.VectorSubcoreMesh(core_axis_name="c", subcore_axis_name="s")


def _prep_body(src_hbm, dst_hbm, w_hbm, zcol_hbm, out_hbm,
               dinv_v, src_v, dst_v, w_v, o_v, dg_v, deg_sh):
  c = lax.axis_index("c")
  s = lax.axis_index("s")
  wid = c * _NS + s

  pltpu.sync_copy(zcol_hbm, deg_sh.at[pl.ds(s * _NROWS_TILE, _NROWS_TILE)])
  plsc.subcore_barrier()

  for q in range(_EROWS_TILE // _QR):
    rq = s * _EROWS_TILE + q * _QR
    pltpu.sync_copy(dst_hbm.at[pl.ds(rq, _QR)], dst_v)
    pltpu.sync_copy(w_hbm.at[pl.ds(rq, _QR)], w_v)

    @pl.loop(0, _QR)
    def _scat(j):
      pltpu.sync_copy(w_v.at[j], deg_sh.at[dst_v.at[j]], add=True)

  plsc.subcore_barrier()

  pltpu.sync_copy(deg_sh.at[pl.ds(s * _NROWS_TILE, _NROWS_TILE)], dg_v)

  @pl.loop(0, _NROWS_TILE // _L)
  def _rsq(i):
    d = dg_v[pl.ds(i * _L, _L)]
    bi = plsc.bitcast(d, _i32)
    y = plsc.bitcast(0x5F3759DF - lax.shift_right_logical(bi, 1), _f32)
    y = y * (1.5 - 0.5 * d * y * y)
    y = y * (1.5 - 0.5 * d * y * y)
    y = y * (1.5 - 0.5 * d * y * y)
    dg_v[pl.ds(i * _L, _L)] = y

  pltpu.sync_copy(dg_v, deg_sh.at[pl.ds(s * _NROWS_TILE, _NROWS_TILE)])
  plsc.subcore_barrier()

  pltpu.sync_copy(deg_sh, dinv_v)
  for q in range(_EROWS_W // _QR):
    rq = wid * _EROWS_W + q * _QR
    pltpu.sync_copy(src_hbm.at[pl.ds(rq, _QR)], src_v)
    pltpu.sync_copy(dst_hbm.at[pl.ds(rq, _QR)], dst_v)
    pltpu.sync_copy(w_hbm.at[pl.ds(rq, _QR)], w_v)

    @pl.loop(0, _QR)
    def _row(r):

      @pl.loop(0, _CHUNK // _L)
      def _grp(g):
        sl = pl.ds(g * _L, _L)
        a = plsc.load_gather(dinv_v, [src_v[r, sl]])
        b = plsc.load_gather(dinv_v, [dst_v[r, sl]])
        o_v[r, sl] = a * b * w_v[r, sl]

    pltpu.sync_copy(o_v, out_hbm.at[pl.ds(rq, _QR)])


def _k123_norm(src2d, dst2d, w2d, zcol):
  kfn = pl.kernel(
      _prep_body,
      out_type=jax.ShapeDtypeStruct((_E2_ROWS, _CHUNK), _f32),
      mesh=_sc_mesh(),
      compiler_params=pltpu.CompilerParams(
          needs_layout_passes=False, use_tc_tiling_on_sc=False),
      scratch_types=[
          pltpu.VMEM((_N_PAD,), _f32),
          pltpu.VMEM((_QR, _CHUNK), _i32),
          pltpu.VMEM((_QR, _CHUNK), _i32),
          pltpu.VMEM((_QR, _CHUNK), _f32),
          pltpu.VMEM((_QR, _CHUNK), _f32),
          pltpu.VMEM((_NROWS_TILE,), _f32),
          pltpu.VMEM_SHARED((_N_PAD,), _f32),
      ],
  )
  return kfn(src2d, dst2d, w2d, zcol)


def _agg_body(src_hbm, dst_hbm, nrm_hbm, zrows_hbm, xt_hbm, out_hbm,
              src_v, dst_v, nrm_v, rb0, rb1, rb2, rb3,
              sb0, sb1, sb2, sb3, gsem, ssem, tsem, ax_sh):
  c = lax.axis_index("c")
  s = lax.axis_index("s")
  rbs = (rb0, rb1, rb2, rb3)
  sbs = (sb0, sb1, sb2, sb3)
  io2 = jnp.arange(_L, dtype=_i32) * 2
  io2p1 = io2 + 1

  def one_period(pp):
    xref = xt_hbm.at[pp]

    def fire_gather(i, k):
      pltpu.async_copy(xref.at[src_v.at[i]], rbs[k], gsem)

    def wait_gather(k):
      pltpu.make_async_copy(xref.at[pl.ds(0, _CHUNK)], rbs[k], gsem).wait()

    def fire_scatter(i, k):
      pltpu.async_copy(sbs[k], ax_sh.at[dst_v.at[i]], ssem, add=True)

    def wait_scatter(k):
      pltpu.make_async_copy(
          sbs[k], ax_sh.at[pl.ds(0, _CHUNK)], ssem).wait()

    pltpu.sync_copy(
        zrows_hbm, ax_sh.at[pl.ds(s * _AROWS_TILE, _AROWS_TILE)])
    plsc.subcore_barrier()

    @pl.loop(0, _NSB)
    def _blk(b):
      r0 = s * _EROWS_TILE + b * _SB
      pltpu.async_copy(src_hbm.at[pl.ds(r0, _SB)], src_v, tsem)
      pltpu.async_copy(dst_hbm.at[pl.ds(r0, _SB)], dst_v, tsem)
      d = pltpu.async_copy(
          nrm_hbm.at[pl.ds(r0 * _CHUNK, _SB * _CHUNK)], nrm_v, tsem)
      pltpu.make_async_copy(src_hbm.at[pl.ds(0, _SB)], src_v, tsem).wait()
      pltpu.make_async_copy(dst_hbm.at[pl.ds(0, _SB)], dst_v, tsem).wait()
      d.wait()

      fire_gather(0, 0)
      fire_gather(1, 1)

      @pl.loop(0, _SB, step=_NRB)
      def _quad(jq):
        for k in range(_NRB):
          i = jq + k
          wait_gather(k)

          @pl.when(i >= 2)
          def _(k=k):
            wait_scatter((k + 2) % _NRB)

          @pl.when(i < _SB - 2)
          def _(i=i, k=k):
            fire_gather(i + 2, (k + 2) % _NRB)

          @plsc.parallel_loop(0, _CHUNK, unroll=8)
          def _scale(e, i=i, k=k):
            nb = plsc.load_gather(nrm_v, [jnp.full((_L,), i * _CHUNK + e, _i32)])
            wi = plsc.bitcast(rbs[k][e, pl.ds(0, 2 * _L)], _i32)
            ve = plsc.bitcast(lax.shift_left(wi, 16), _f32) * nb
            vo = plsc.bitcast(
                jnp.bitwise_and(wi, jnp.int32(-65536)), _f32) * nb
            er = jnp.full((_L,), e, _i32)
            plsc.store_scatter(sbs[k], [er, io2], ve)
            plsc.store_scatter(sbs[k], [er, io2p1], vo)

          fire_scatter(i, k)

      wait_scatter(_NRB - 2)
      wait_scatter(_NRB - 1)

    plsc.subcore_barrier()
    pltpu.sync_copy(
        ax_sh.at[pl.ds(s * _AROWS_TILE, _AROWS_TILE)],
        out_hbm.at[pp].at[pl.ds(s * _AROWS_TILE, _AROWS_TILE)])

  for ci in range(_NC):

    @pl.when(c == ci)
    def _(ci=ci):
      for lp in range(_PERIODS // _NC):
        one_period(ci * (_PERIODS // _NC) + lp)


def _k4_agg(src2d, dst2d, nrm1d, zrows, xt):
  kfn = pl.kernel(
      _agg_body,
      out_type=jax.ShapeDtypeStruct((_PERIODS, _N_PAD_AX, _IN_C), _f32),
      mesh=_sc_mesh(),
      compiler_params=pltpu.CompilerParams(
          needs_layout_passes=False, use_tc_tiling_on_sc=False),
      scratch_types=[
          pltpu.VMEM((_SB, _CHUNK), _i32),
          pltpu.VMEM((_SB, _CHUNK), _i32),
          pltpu.VMEM((_SB * _CHUNK,), _f32),
          pltpu.VMEM((_CHUNK, _IN_C), jnp.bfloat16),
          pltpu.VMEM((_CHUNK, _IN_C), jnp.bfloat16),
          pltpu.VMEM((_CHUNK, _IN_C), jnp.bfloat16),
          pltpu.VMEM((_CHUNK, _IN_C), jnp.bfloat16),
          pltpu.VMEM((_CHUNK, _IN_C), _f32),
          pltpu.VMEM((_CHUNK, _IN_C), _f32),
          pltpu.VMEM((_CHUNK, _IN_C), _f32),
          pltpu.VMEM((_CHUNK, _IN_C), _f32),
          pltpu.SemaphoreType.DMA,
          pltpu.SemaphoreType.DMA,
          pltpu.SemaphoreType.DMA,
          pltpu.VMEM_SHARED((_N_PAD_AX, _IN_C), _f32),
      ],
  )
  return kfn(src2d, dst2d, nrm1d, zrows, xt)


_BN4 = 512


def _dense_body(axs_ref, probs_ref, wz_ref, bz_ref, wh_ref, bh_ref, out_ref):
  acc = jnp.zeros((_BN4, 4 * _OUT_C), _f32)
  for p in range(_PERIODS):
    m = axs_ref[p]
    z = jax.nn.sigmoid(
        jnp.dot(m, wz_ref[...], preferred_element_type=_f32) + bz_ref[...])
    t = jnp.tanh(
        jnp.dot(m, wh_ref[...], preferred_element_type=_f32) + bh_ref[...])
    acc = acc + probs_ref[p] * (1.0 - z) * t
  out_ref[...] = acc


def _k5_dense(axs4, probs, wz4, bz4, wh4, bh4):
  grid = ((_N // 4 + _BN4 - 1) // _BN4,)
  return pl.pallas_call(
      _dense_body,
      grid=grid,
      in_specs=[
          pl.BlockSpec((_PERIODS, _BN4, 4 * _IN_C), lambda i: (0, i, 0)),
          pl.BlockSpec(memory_space=pltpu.SMEM),
          pl.BlockSpec((4 * _IN_C, 4 * _OUT_C), lambda i: (0, 0)),
          pl.BlockSpec((1, 4 * _OUT_C), lambda i: (0, 0)),
          pl.BlockSpec((4 * _IN_C, 4 * _OUT_C), lambda i: (0, 0)),
          pl.BlockSpec((1, 4 * _OUT_C), lambda i: (0, 0)),
      ],
      out_specs=pl.BlockSpec((_BN4, 4 * _OUT_C), lambda i: (i, 0)),
      out_shape=jax.ShapeDtypeStruct((_N // 4, 4 * _OUT_C), _f32),
  )(axs4, probs, wz4, bz4, wh4, bh4)


def kernel(X, edge_index, edge_weight, attention,
           Wc_z, bc_z, Wl_z, bl_z,
           Wc_r, bc_r, Wl_r, bl_r,
           Wc_h, bc_h, Wl_h, bl_h):
  src = edge_index[0]
  dst = edge_index[1]
  loop_idx = jnp.arange(_N, dtype=_i32)
  pad = _E2_PAD - _E2
  src2 = jnp.concatenate([src, loop_idx, jnp.zeros((pad,), _i32)])
  dst2 = jnp.concatenate([dst, loop_idx, jnp.zeros((pad,), _i32)])
  w2 = jnp.concatenate(
      [edge_weight, jnp.ones((_N,), _f32), jnp.zeros((pad,), _f32)])
  src2d = src2.reshape(_E2_ROWS, _CHUNK)
  dst2d = dst2.reshape(_E2_ROWS, _CHUNK)
  w2d = w2.reshape(_E2_ROWS, _CHUNK)

  Xt = jnp.transpose(X.astype(jnp.bfloat16), (2, 0, 1))

  wz = Wc_z @ Wl_z[:_OUT_C]
  bz = (bc_z @ Wl_z[:_OUT_C] + bl_z).reshape(1, _OUT_C)
  wh = Wc_h @ Wl_h[:_OUT_C]
  bh = (bc_h @ Wl_h[:_OUT_C] + bl_h).reshape(1, _OUT_C)
  probs = jax.nn.softmax(attention, axis=0)

  zcol = jnp.zeros((_NROWS_TILE,), _f32)
  nrm2d = _k123_norm(src2d, dst2d, w2d, zcol)
  zrows = jnp.zeros((_AROWS_TILE, _IN_C), _f32)
  axs = _k4_agg(src2d, dst2d, nrm2d.reshape(_E2_PAD), zrows, Xt)

  axs4 = axs.reshape(_PERIODS, _N_PAD_AX // 4, 4 * _IN_C)
  zeros_blk = jnp.zeros((_IN_C, _OUT_C), _f32)

  def blockdiag4(w):
    rows = []
    for r in range(4):
      rows.append(jnp.concatenate(
          [w if cc == r else zeros_blk for cc in range(4)], axis=1))
    return jnp.concatenate(rows, axis=0)

  wz4 = blockdiag4(wz)
  wh4 = blockdiag4(wh)
  bz4 = jnp.tile(bz, (1, 4))
  bh4 = jnp.tile(bh, (1, 4))
  out4 = _k5_dense(axs4, probs, wz4, bz4, wh4, bh4)
  return out4.reshape(_N, _OUT_C)


# speedup vs baseline: 1.5247x; 1.5247x over previous
import functools

import jax
import jax.numpy as jnp
from jax import lax
from jax.experimental import pallas as pl
from jax.experimental.pallas import tpu as pltpu
from jax.experimental.pallas import tpu_sc as plsc

_N = 50000
_E = 800000
_IN_C = 32
_OUT_C = 64
_PERIODS = 8

_NC, _NS, _L = 2, 16, 16
_NW = _NC * _NS

_N_PAD = 51200
_NROWS_TILE = _N_PAD // _NS
_E2 = _E + _N
_CHUNK = 128
_E2_ROWS = 6656
_E2_PAD = _E2_ROWS * _CHUNK
_EROWS_W = _E2_ROWS // _NW
_EROWS_TILE = _E2_ROWS // _NS
_HALF = _EROWS_TILE // 2
_ZROWS = 200
_QR = 104

_N_PAD_AX = 50176
_AROWS_TILE = _N_PAD_AX // _NS
_SB = 32
_NSB = _EROWS_TILE // _SB
_AZROWS = 112
_NRB = 4

_f32 = jnp.float32
_i32 = jnp.int32


def _sc_mesh():
  return plsc.VectorSubcoreMesh(core_axis_name="c", subcore_axis_name="s")


def _prep_body(src_hbm, dst_hbm, w_hbm, zcol_hbm, out_hbm,
               dinv_v, src_v, dst_v, w_v, o_v, dg_v, deg_sh):
  c = lax.axis_index("c")
  s = lax.axis_index("s")
  wid = c * _NS + s

  pltpu.sync_copy(zcol_hbm, deg_sh.at[pl.ds(s * _NROWS_TILE, _NROWS_TILE)])
  plsc.subcore_barrier()

  for q in range(_EROWS_TILE // _QR):
    rq = s * _EROWS_TILE + q * _QR
    pltpu.sync_copy(dst_hbm.at[pl.ds(rq, _QR)], dst_v)
    pltpu.sync_copy(w_hbm.at[pl.ds(rq, _QR)], w_v)

    @pl.loop(0, _QR)
    def _scat(j):
      pltpu.sync_copy(w_v.at[j], deg_sh.at[dst_v.at[j]], add=True)

  plsc.subcore_barrier()

  pltpu.sync_copy(deg_sh.at[pl.ds(s * _NROWS_TILE, _NROWS_TILE)], dg_v)

  @pl.loop(0, _NROWS_TILE // _L)
  def _rsq(i):
    d = dg_v[pl.ds(i * _L, _L)]
    bi = plsc.bitcast(d, _i32)
    y = plsc.bitcast(0x5F3759DF - lax.shift_right_logical(bi, 1), _f32)
    y = y * (1.5 - 0.5 * d * y * y)
    y = y * (1.5 - 0.5 * d * y * y)
    y = y * (1.5 - 0.5 * d * y * y)
    dg_v[pl.ds(i * _L, _L)] = y

  pltpu.sync_copy(dg_v, deg_sh.at[pl.ds(s * _NROWS_TILE, _NROWS_TILE)])
  plsc.subcore_barrier()

  pltpu.sync_copy(deg_sh, dinv_v)
  for q in range(_EROWS_W // _QR):
    rq = wid * _EROWS_W + q * _QR
    pltpu.sync_copy(src_hbm.at[pl.ds(rq, _QR)], src_v)
    pltpu.sync_copy(dst_hbm.at[pl.ds(rq, _QR)], dst_v)
    pltpu.sync_copy(w_hbm.at[pl.ds(rq, _QR)], w_v)

    @pl.loop(0, _QR)
    def _row(r):

      @pl.loop(0, _CHUNK // _L)
      def _grp(g):
        sl = pl.ds(g * _L, _L)
        a = plsc.load_gather(dinv_v, [src_v[r, sl]])
        b = plsc.load_gather(dinv_v, [dst_v[r, sl]])
        o_v[r, sl] = a * b * w_v[r, sl]

    pltpu.sync_copy(o_v, out_hbm.at[pl.ds(rq, _QR)])


def _k123_norm(src2d, dst2d, w2d, zcol):
  kfn = pl.kernel(
      _prep_body,
      out_type=jax.ShapeDtypeStruct((_E2_ROWS, _CHUNK), _f32),
      mesh=_sc_mesh(),
      compiler_params=pltpu.CompilerParams(
          needs_layout_passes=False, use_tc_tiling_on_sc=False),
      scratch_types=[
          pltpu.VMEM((_N_PAD,), _f32),
          pltpu.VMEM((_QR, _CHUNK), _i32),
          pltpu.VMEM((_QR, _CHUNK), _i32),
          pltpu.VMEM((_QR, _CHUNK), _f32),
          pltpu.VMEM((_QR, _CHUNK), _f32),
          pltpu.VMEM((_NROWS_TILE,), _f32),
          pltpu.VMEM_SHARED((_N_PAD,), _f32),
      ],
  )
  return kfn(src2d, dst2d, w2d, zcol)


def _agg_body(src_hbm, dst_hbm, nrm_hbm, zrows_hbm, xt_hbm, out_hbm,
              src_v, dst_v, nrm_v, rb0, rb1, rb2, rb3,
              gsem, ssem, tsem, ax_sh):
  c = lax.axis_index("c")
  s = lax.axis_index("s")
  rbs = (rb0, rb1, rb2, rb3)

  def one_period(pp):
    xref = xt_hbm.at[pp]

    def fire_gather(i, k):
      pltpu.async_copy(xref.at[src_v.at[i]], rbs[k], gsem)

    def wait_gather(k):
      pltpu.make_async_copy(xref.at[pl.ds(0, _CHUNK)], rbs[k], gsem).wait()

    def fire_scatter(i, k):
      pltpu.async_copy(rbs[k], ax_sh.at[dst_v.at[i]], ssem, add=True)

    def wait_scatter(k):
      pltpu.make_async_copy(
          rbs[k], ax_sh.at[pl.ds(0, _CHUNK)], ssem).wait()

    pltpu.sync_copy(
        zrows_hbm, ax_sh.at[pl.ds(s * _AROWS_TILE, _AROWS_TILE)])
    plsc.subcore_barrier()

    @pl.loop(0, _NSB)
    def _blk(b):
      r0 = s * _EROWS_TILE + b * _SB
      pltpu.async_copy(src_hbm.at[pl.ds(r0, _SB)], src_v, tsem)
      pltpu.async_copy(dst_hbm.at[pl.ds(r0, _SB)], dst_v, tsem)
      d = pltpu.async_copy(
          nrm_hbm.at[pl.ds(r0 * _CHUNK, _SB * _CHUNK)], nrm_v, tsem)
      pltpu.make_async_copy(src_hbm.at[pl.ds(0, _SB)], src_v, tsem).wait()
      pltpu.make_async_copy(dst_hbm.at[pl.ds(0, _SB)], dst_v, tsem).wait()
      d.wait()

      fire_gather(0, 0)
      fire_gather(1, 1)

      @pl.loop(0, _SB, step=_NRB)
      def _quad(jq):
        for k in range(_NRB):
          i = jq + k
          wait_gather(k)

          @pl.when(i >= 2)
          def _(k=k):
            wait_scatter((k + 2) % _NRB)

          @pl.when(i < _SB - 2)
          def _(i=i, k=k):
            fire_gather(i + 2, (k + 2) % _NRB)

          @plsc.parallel_loop(0, _CHUNK, unroll=8)
          def _scale(e, i=i, k=k):
            nb = plsc.load_gather(nrm_v, [jnp.full((_L,), i * _CHUNK + e, _i32)])
            rbs[k][e, pl.ds(0, _L)] = rbs[k][e, pl.ds(0, _L)] * nb
            rbs[k][e, pl.ds(_L, _L)] = rbs[k][e, pl.ds(_L, _L)] * nb

          fire_scatter(i, k)

      wait_scatter(_NRB - 2)
      wait_scatter(_NRB - 1)

    plsc.subcore_barrier()
    pltpu.sync_copy(
        ax_sh.at[pl.ds(s * _AROWS_TILE, _AROWS_TILE)],
        out_hbm.at[pp].at[pl.ds(s * _AROWS_TILE, _AROWS_TILE)])

  for ci in range(_NC):

    @pl.when(c == ci)
    def _(ci=ci):
      for lp in range(_PERIODS // _NC):
        one_period(ci * (_PERIODS // _NC) + lp)


def _k4_agg(src2d, dst2d, nrm1d, zrows, xt):
  kfn = pl.kernel(
      _agg_body,
      out_type=jax.ShapeDtypeStruct((_PERIODS, _N_PAD_AX, _IN_C), _f32),
      mesh=_sc_mesh(),
      compiler_params=pltpu.CompilerParams(
          needs_layout_passes=False, use_tc_tiling_on_sc=False),
      scratch_types=[
          pltpu.VMEM((_SB, _CHUNK), _i32),
          pltpu.VMEM((_SB, _CHUNK), _i32),
          pltpu.VMEM((_SB * _CHUNK,), _f32),
          pltpu.VMEM((_CHUNK, _IN_C), _f32),
          pltpu.VMEM((_CHUNK, _IN_C), _f32),
          pltpu.VMEM((_CHUNK, _IN_C), _f32),
          pltpu.VMEM((_CHUNK, _IN_C), _f32),
          pltpu.SemaphoreType.DMA,
          pltpu.SemaphoreType.DMA,
          pltpu.SemaphoreType.DMA,
          pltpu.VMEM_SHARED((_N_PAD_AX, _IN_C), _f32),
      ],
  )
  return kfn(src2d, dst2d, nrm1d, zrows, xt)


_BN4 = 512


def _dense_body(axs_ref, probs_ref, wz_ref, bz_ref, wh_ref, bh_ref, out_ref):
  acc = jnp.zeros((_BN4, 4 * _OUT_C), _f32)
  for p in range(_PERIODS):
    m = axs_ref[p]
    z = jax.nn.sigmoid(
        jnp.dot(m, wz_ref[...], preferred_element_type=_f32) + bz_ref[...])
    t = jnp.tanh(
        jnp.dot(m, wh_ref[...], preferred_element_type=_f32) + bh_ref[...])
    acc = acc + probs_ref[p] * (1.0 - z) * t
  out_ref[...] = acc


def _k5_dense(axs4, probs, wz4, bz4, wh4, bh4):
  grid = ((_N // 4 + _BN4 - 1) // _BN4,)
  return pl.pallas_call(
      _dense_body,
      grid=grid,
      in_specs=[
          pl.BlockSpec((_PERIODS, _BN4, 4 * _IN_C), lambda i: (0, i, 0)),
          pl.BlockSpec(memory_space=pltpu.SMEM),
          pl.BlockSpec((4 * _IN_C, 4 * _OUT_C), lambda i: (0, 0)),
          pl.BlockSpec((1, 4 * _OUT_C), lambda i: (0, 0)),
          pl.BlockSpec((4 * _IN_C, 4 * _OUT_C), lambda i: (0, 0)),
          pl.BlockSpec((1, 4 * _OUT_C), lambda i: (0, 0)),
      ],
      out_specs=pl.BlockSpec((_BN4, 4 * _OUT_C), lambda i: (i, 0)),
      out_shape=jax.ShapeDtypeStruct((_N // 4, 4 * _OUT_C), _f32),
  )(axs4, probs, wz4, bz4, wh4, bh4)


def kernel(X, edge_index, edge_weight, attention,
           Wc_z, bc_z, Wl_z, bl_z,
           Wc_r, bc_r, Wl_r, bl_r,
           Wc_h, bc_h, Wl_h, bl_h):
  src = edge_index[0]
  dst = edge_index[1]
  loop_idx = jnp.arange(_N, dtype=_i32)
  pad = _E2_PAD - _E2
  src2 = jnp.concatenate([src, loop_idx, jnp.zeros((pad,), _i32)])
  dst2 = jnp.concatenate([dst, loop_idx, jnp.zeros((pad,), _i32)])
  w2 = jnp.concatenate(
      [edge_weight, jnp.ones((_N,), _f32), jnp.zeros((pad,), _f32)])
  src2d = src2.reshape(_E2_ROWS, _CHUNK)
  dst2d = dst2.reshape(_E2_ROWS, _CHUNK)
  w2d = w2.reshape(_E2_ROWS, _CHUNK)

  Xt = jnp.transpose(X, (2, 0, 1))

  wz = Wc_z @ Wl_z[:_OUT_C]
  bz = (bc_z @ Wl_z[:_OUT_C] + bl_z).reshape(1, _OUT_C)
  wh = Wc_h @ Wl_h[:_OUT_C]
  bh = (bc_h @ Wl_h[:_OUT_C] + bl_h).reshape(1, _OUT_C)
  probs = jax.nn.softmax(attention, axis=0)

  zcol = jnp.zeros((_NROWS_TILE,), _f32)
  nrm2d = _k123_norm(src2d, dst2d, w2d, zcol)
  zrows = jnp.zeros((_AROWS_TILE, _IN_C), _f32)
  axs = _k4_agg(src2d, dst2d, nrm2d.reshape(_E2_PAD), zrows, Xt)

  axs4 = axs.reshape(_PERIODS, _N_PAD_AX // 4, 4 * _IN_C)
  zeros_blk = jnp.zeros((_IN_C, _OUT_C), _f32)

  def blockdiag4(w):
    rows = []
    for r in range(4):
      rows.append(jnp.concatenate(
          [w if cc == r else zeros_blk for cc in range(4)], axis=1))
    return jnp.concatenate(rows, axis=0)

  wz4 = blockdiag4(wz)
  wh4 = blockdiag4(wh)
  bz4 = jnp.tile(bz, (1, 4))
  bh4 = jnp.tile(bh, (1, 4))
  out4 = _k5_dense(axs4, probs, wz4, bz4, wh4, bh4)
  return out4.reshape(_N, _OUT_C)
